# baseline (speedup 1.0000x reference)
import functools

import jax
import jax.numpy as jnp
from jax import lax
from jax.experimental import pallas as pl
from jax.experimental.pallas import tpu as pltpu
from jax.experimental.pallas import tpu_sc as plsc

N = 10000
E = 320000
G = 128
D_OUT = 10

NTILES = 16
LANES = 16
N_PAD = 10240
E_PER_TILE = E // NTILES
GROUPS = E_PER_TILE // LANES
STRIP = N_PAD // NTILES
SGROUPS = STRIP // LANES


def _rsqrt16(d):
    i = plsc.bitcast(d, jnp.int32)
    i = jnp.int32(0x5F3759DF) - lax.shift_right_logical(i, 1)
    y = plsc.bitcast(i, jnp.float32)
    for _ in range(3):
        y = y * (1.5 - 0.5 * d * y * y)
    return jnp.where(d > 0.0, y, 0.0)


def _sc_body(x_hbm, ei_hbm, tx1_hbm, tx2_hbm,
             src_v, dst_v, dinv_v, aux_v, acc_v, rbuf_v,
             strip_v, dstrip_v, xstrip_v, sem,
             sh_all, sh_node, sh_node2):
    t = lax.axis_index("s")
    e0 = t * E_PER_TILE
    cp1 = pltpu.async_copy(ei_hbm.at[pl.ds(e0, E_PER_TILE)], src_v, sem)
    cp2 = pltpu.async_copy(ei_hbm.at[pl.ds(E + e0, E_PER_TILE)], dst_v, sem)
    cp3 = pltpu.async_copy(x_hbm.at[pl.ds(t * STRIP, STRIP)], xstrip_v, sem)

    zeros = jnp.zeros((LANES,), jnp.float32)

    def zero_acc():
        @plsc.parallel_loop(0, N_PAD, LANES, unroll=8)
        def _(i):
            acc_v[pl.ds(i, LANES)] = zeros

    def publish_and_reduce_strip():
        pltpu.sync_copy(acc_v, sh_all.at[pl.ds(t * N_PAD, N_PAD)])
        plsc.subcore_barrier()
        cps = [pltpu.async_copy(
                   sh_all.at[pl.ds(j * N_PAD + t * STRIP, STRIP)],
                   rbuf_v.at[pl.ds(j * STRIP, STRIP)], sem)
               for j in range(NTILES)]
        for cp in cps:
            cp.wait()

        @plsc.parallel_loop(0, STRIP, LANES, unroll=4)
        def _(i):
            v = rbuf_v[pl.ds(i, LANES)]
            for j in range(1, NTILES):
                v = v + rbuf_v[pl.ds(j * STRIP + i, LANES)]
            strip_v[pl.ds(i, LANES)] = v

    zero_acc()
    cp1.wait()
    cp2.wait()
    cp3.wait()
    ones = jnp.full((LANES,), 1.0, jnp.float32)

    @plsc.parallel_loop(0, E_PER_TILE, LANES, unroll=8)
    def _(i):
        s = src_v[pl.ds(i, LANES)]
        plsc.addupdate_scatter(acc_v, [s], ones)

    publish_and_reduce_strip()

    @plsc.parallel_loop(0, STRIP, LANES, unroll=4)
    def _(i):
        dv = _rsqrt16(strip_v[pl.ds(i, LANES)])
        dstrip_v[pl.ds(i, LANES)] = dv
        strip_v[pl.ds(i, LANES)] = dv * xstrip_v[pl.ds(i, LANES)]

    pltpu.sync_copy(dstrip_v, sh_node2.at[pl.ds(t * STRIP, STRIP)])
    pltpu.sync_copy(strip_v, sh_node.at[pl.ds(t * STRIP, STRIP)])
    plsc.subcore_barrier()
    cp4 = pltpu.async_copy(sh_node2, dinv_v, sem)
    cp5 = pltpu.async_copy(sh_node, aux_v, sem)

    zero_acc()
    cp4.wait()
    cp5.wait()

    @plsc.parallel_loop(0, E_PER_TILE, LANES, unroll=8)
    def _(i):
        s = src_v[pl.ds(i, LANES)]
        d = dst_v[pl.ds(i, LANES)]
        a = plsc.load_gather(aux_v, [s])
        b = plsc.load_gather(dinv_v, [d])
        plsc.addupdate_scatter(acc_v, [d], a * b)

    publish_and_reduce_strip()

    @plsc.parallel_loop(0, STRIP, LANES, unroll=4)
    def _(i):
        v = -strip_v[pl.ds(i, LANES)]
        strip_v[pl.ds(i, LANES)] = v
        dstrip_v[pl.ds(i, LANES)] = dstrip_v[pl.ds(i, LANES)] * v

    cp6 = pltpu.async_copy(strip_v, tx1_hbm.at[pl.ds(t * STRIP, STRIP)], sem)
    pltpu.sync_copy(dstrip_v, sh_node.at[pl.ds(t * STRIP, STRIP)])
    plsc.subcore_barrier()
    cp7 = pltpu.async_copy(sh_node, aux_v, sem)

    zero_acc()
    cp6.wait()
    cp7.wait()

    @plsc.parallel_loop(0, E_PER_TILE, LANES, unroll=8)
    def _(i):
        s = src_v[pl.ds(i, LANES)]
        d = dst_v[pl.ds(i, LANES)]
        a = plsc.load_gather(aux_v, [s])
        b = plsc.load_gather(dinv_v, [d])
        plsc.addupdate_scatter(acc_v, [d], a * b)

    publish_and_reduce_strip()

    @plsc.parallel_loop(0, STRIP, LANES, unroll=4)
    def _(i):
        strip_v[pl.ds(i, LANES)] = -strip_v[pl.ds(i, LANES)]

    pltpu.sync_copy(strip_v, tx2_hbm.at[pl.ds(t * STRIP, STRIP)])


@functools.cache
def _get_sc_prop():
    return pl.kernel(
        _sc_body,
        out_type=[jax.ShapeDtypeStruct((N_PAD,), jnp.float32),
                  jax.ShapeDtypeStruct((N_PAD,), jnp.float32)],
        mesh=plsc.VectorSubcoreMesh(
            core_axis_name="c", subcore_axis_name="s",
            num_cores=1, num_subcores=NTILES),
        scratch_types=[
            pltpu.VMEM((E_PER_TILE,), jnp.int32),
            pltpu.VMEM((E_PER_TILE,), jnp.int32),
            pltpu.VMEM((N_PAD,), jnp.float32),
            pltpu.VMEM((N_PAD,), jnp.float32),
            pltpu.VMEM((N_PAD,), jnp.float32),
            pltpu.VMEM((N_PAD,), jnp.float32),
            pltpu.VMEM((STRIP,), jnp.float32),
            pltpu.VMEM((STRIP,), jnp.float32),
            pltpu.VMEM((STRIP,), jnp.float32),
            pltpu.SemaphoreType.DMA,
            pltpu.VMEM_SHARED((NTILES * N_PAD,), jnp.float32),
            pltpu.VMEM_SHARED((N_PAD,), jnp.float32),
            pltpu.VMEM_SHARED((N_PAD,), jnp.float32),
        ],
        compiler_params=pltpu.CompilerParams(needs_layout_passes=False),
    )


BN = 2000
NG = N * G
BF = 2500
CB = BF * G


def _hmat_body(t_ref, wc_ref, bch_ref, h_ref):
    h = (t_ref[:, 0:1] * wc_ref[0:1, :]
         + t_ref[:, 1:2] * wc_ref[1:2, :]
         + t_ref[:, 2:3] * wc_ref[2:3, :]
         + bch_ref[0:1, :])
    h_ref[...] = jnp.maximum(h, 0.0)


_hmat = pl.pallas_call(
    _hmat_body,
    grid=(N // BN,),
    in_specs=[
        pl.BlockSpec((BN, 3), lambda i: (i, 0)),
        pl.BlockSpec((3, G), lambda i: (0, 0)),
        pl.BlockSpec((1, G), lambda i: (0, 0)),
    ],
    out_specs=pl.BlockSpec((BN, G), lambda i: (i, 0)),
    out_shape=jax.ShapeDtypeStruct((N, G), jnp.float32),
)


def _fc_body(w_ref, h_ref, bfc_ref, out_ref, acc_ref):
    i = pl.program_id(0)

    @pl.when(i == 0)
    def _():
        acc_ref[...] = bfc_ref[...]

    prod = w_ref[...] * h_ref[...]
    acc_ref[...] += jnp.sum(prod, axis=1, keepdims=True)

    @pl.when(i == pl.num_programs(0) - 1)
    def _():
        y = acc_ref[...]
        m = jnp.max(y)
        s = jnp.sum(jnp.exp(y - m))
        out_ref[...] = y - m - jnp.log(jnp.broadcast_to(s, (D_OUT, 1)))


_fc = pl.pallas_call(
    _fc_body,
    grid=(NG // CB,),
    in_specs=[
        pl.BlockSpec((D_OUT, CB), lambda i: (0, i)),
        pl.BlockSpec((1, CB), lambda i: (0, i)),
        pl.BlockSpec((D_OUT, 1), lambda i: (0, 0)),
    ],
    out_specs=pl.BlockSpec((D_OUT, 1), lambda i: (0, 0)),
    out_shape=jax.ShapeDtypeStruct((D_OUT, 1), jnp.float32),
    scratch_shapes=[pltpu.VMEM((D_OUT, 1), jnp.float32)],
)


def kernel(x, edge_index, W_cheb, b_cheb, W_fc, b_fc):
    x0 = x[:, 0]
    ei_flat = edge_index.reshape(2 * E)
    x_pad = jnp.concatenate([x0, jnp.zeros((N_PAD - N,), jnp.float32)])
    tx1_p, tx2_p = _get_sc_prop()(x_pad, ei_flat)
    tx1 = tx1_p[:N]
    tx2h = tx2_p[:N]
    t_mat = jnp.stack([x0, tx1, tx2h], axis=1)
    wc = W_cheb.reshape(3, G)
    wc_eff = jnp.stack([wc[0] - wc[2], wc[1], 2.0 * wc[2]], axis=0)
    bch = b_cheb.reshape(1, G)
    h = _hmat(t_mat, wc_eff, bch)
    h_row = h.reshape(1, NG)
    yv = _fc(W_fc, h_row, b_fc.reshape(D_OUT, 1))
    return yv[:, 0]


# speedup vs baseline: 1.0139x; 1.0139x over previous
import functools

import jax
import jax.numpy as jnp
from jax import lax
from jax.experimental import pallas as pl
from jax.experimental.pallas import tpu as pltpu
from jax.experimental.pallas import tpu_sc as plsc

N = 10000
E = 320000
G = 128
D_OUT = 10

NTILES = 16
LANES = 16
N_PAD = 10240
E_PER_TILE = E // NTILES
GROUPS = E_PER_TILE // LANES
STRIP = N_PAD // NTILES
SGROUPS = STRIP // LANES


def _rsqrt16(d):
    i = plsc.bitcast(d, jnp.int32)
    i = jnp.int32(0x5F3759DF) - lax.shift_right_logical(i, 1)
    y = plsc.bitcast(i, jnp.float32)
    for _ in range(3):
        y = y * (1.5 - 0.5 * d * y * y)
    return jnp.where(d > 0.0, y, 0.0)


def _sc_body(x_hbm, ei_hbm, tx1_hbm, tx2_hbm,
             src_v, dst_v, dinv_v, aux_v, acc_v, rbuf_v,
             strip_v, dstrip_v, xstrip_v, sem,
             sh_all, sh_node, sh_node2):
    t = lax.axis_index("s")
    e0 = t * E_PER_TILE
    cp1 = pltpu.async_copy(ei_hbm.at[pl.ds(e0, E_PER_TILE)], src_v, sem)
    cp2 = pltpu.async_copy(ei_hbm.at[pl.ds(E + e0, E_PER_TILE)], dst_v, sem)
    cp3 = pltpu.async_copy(x_hbm.at[pl.ds(t * STRIP, STRIP)], xstrip_v, sem)

    zeros = jnp.zeros((LANES,), jnp.float32)

    def zero_acc():
        @plsc.parallel_loop(0, N_PAD, LANES, unroll=8)
        def _(i):
            acc_v[pl.ds(i, LANES)] = zeros

    def publish_and_reduce_strip():
        pltpu.sync_copy(acc_v, sh_all.at[pl.ds(t * N_PAD, N_PAD)])
        plsc.subcore_barrier()
        cps = [pltpu.async_copy(
                   sh_all.at[pl.ds(j * N_PAD + t * STRIP, STRIP)],
                   rbuf_v.at[pl.ds(j * STRIP, STRIP)], sem)
               for j in range(NTILES)]
        for cp in cps:
            cp.wait()

        @plsc.parallel_loop(0, STRIP, LANES, unroll=4)
        def _(i):
            v = rbuf_v[pl.ds(i, LANES)]
            for j in range(1, NTILES):
                v = v + rbuf_v[pl.ds(j * STRIP + i, LANES)]
            strip_v[pl.ds(i, LANES)] = v

    zero_acc()
    cp1.wait()
    cp2.wait()
    cp3.wait()
    ones = jnp.full((LANES,), 1.0, jnp.float32)

    @plsc.parallel_loop(0, E_PER_TILE, LANES, unroll=8)
    def _(i):
        s = src_v[pl.ds(i, LANES)]
        plsc.addupdate_scatter(acc_v, [s], ones)

    publish_and_reduce_strip()

    @plsc.parallel_loop(0, STRIP, LANES, unroll=4)
    def _(i):
        dv = _rsqrt16(strip_v[pl.ds(i, LANES)])
        dstrip_v[pl.ds(i, LANES)] = dv
        strip_v[pl.ds(i, LANES)] = dv * xstrip_v[pl.ds(i, LANES)]

    pltpu.sync_copy(dstrip_v, sh_node2.at[pl.ds(t * STRIP, STRIP)])
    pltpu.sync_copy(strip_v, sh_node.at[pl.ds(t * STRIP, STRIP)])
    plsc.subcore_barrier()
    cp4 = pltpu.async_copy(sh_node2, dinv_v, sem)
    cp5 = pltpu.async_copy(sh_node, aux_v, sem)

    zero_acc()
    cp4.wait()
    cp5.wait()

    @plsc.parallel_loop(0, E_PER_TILE, LANES, unroll=16)
    def _(i):
        s = src_v[pl.ds(i, LANES)]
        d = dst_v[pl.ds(i, LANES)]
        a = plsc.load_gather(aux_v, [s])
        b = plsc.load_gather(dinv_v, [d])
        plsc.addupdate_scatter(acc_v, [d], a * b)

    publish_and_reduce_strip()

    @plsc.parallel_loop(0, STRIP, LANES, unroll=4)
    def _(i):
        v = -strip_v[pl.ds(i, LANES)]
        strip_v[pl.ds(i, LANES)] = v
        dstrip_v[pl.ds(i, LANES)] = dstrip_v[pl.ds(i, LANES)] * v

    cp6 = pltpu.async_copy(strip_v, tx1_hbm.at[pl.ds(t * STRIP, STRIP)], sem)
    pltpu.sync_copy(dstrip_v, sh_node.at[pl.ds(t * STRIP, STRIP)])
    plsc.subcore_barrier()
    cp7 = pltpu.async_copy(sh_node, aux_v, sem)

    zero_acc()
    cp6.wait()
    cp7.wait()

    @plsc.parallel_loop(0, E_PER_TILE, LANES, unroll=16)
    def _(i):
        s = src_v[pl.ds(i, LANES)]
        d = dst_v[pl.ds(i, LANES)]
        a = plsc.load_gather(aux_v, [s])
        b = plsc.load_gather(dinv_v, [d])
        plsc.addupdate_scatter(acc_v, [d], a * b)

    publish_and_reduce_strip()

    @plsc.parallel_loop(0, STRIP, LANES, unroll=4)
    def _(i):
        strip_v[pl.ds(i, LANES)] = -strip_v[pl.ds(i, LANES)]

    pltpu.sync_copy(strip_v, tx2_hbm.at[pl.ds(t * STRIP, STRIP)])


@functools.cache
def _get_sc_prop():
    return pl.kernel(
        _sc_body,
        out_type=[jax.ShapeDtypeStruct((N_PAD,), jnp.float32),
                  jax.ShapeDtypeStruct((N_PAD,), jnp.float32)],
        mesh=plsc.VectorSubcoreMesh(
            core_axis_name="c", subcore_axis_name="s",
            num_cores=1, num_subcores=NTILES),
        scratch_types=[
            pltpu.VMEM((E_PER_TILE,), jnp.int32),
            pltpu.VMEM((E_PER_TILE,), jnp.int32),
            pltpu.VMEM((N_PAD,), jnp.float32),
            pltpu.VMEM((N_PAD,), jnp.float32),
            pltpu.VMEM((N_PAD,), jnp.float32),
            pltpu.VMEM((N_PAD,), jnp.float32),
            pltpu.VMEM((STRIP,), jnp.float32),
            pltpu.VMEM((STRIP,), jnp.float32),
            pltpu.VMEM((STRIP,), jnp.float32),
            pltpu.SemaphoreType.DMA,
            pltpu.VMEM_SHARED((NTILES * N_PAD,), jnp.float32),
            pltpu.VMEM_SHARED((N_PAD,), jnp.float32),
            pltpu.VMEM_SHARED((N_PAD,), jnp.float32),
        ],
        compiler_params=pltpu.CompilerParams(needs_layout_passes=False),
    )


BN = 2000
NG = N * G
BF = 1000
CB = BF * G


def _hmat_body(t_ref, wc_ref, bch_ref, h_ref):
    h = (t_ref[:, 0:1] * wc_ref[0:1, :]
         + t_ref[:, 1:2] * wc_ref[1:2, :]
         + t_ref[:, 2:3] * wc_ref[2:3, :]
         + bch_ref[0:1, :])
    h_ref[...] = jnp.maximum(h, 0.0)


_hmat = pl.pallas_call(
    _hmat_body,
    grid=(N // BN,),
    in_specs=[
        pl.BlockSpec((BN, 3), lambda i: (i, 0)),
        pl.BlockSpec((3, G), lambda i: (0, 0)),
        pl.BlockSpec((1, G), lambda i: (0, 0)),
    ],
    out_specs=pl.BlockSpec((BN, G), lambda i: (i, 0)),
    out_shape=jax.ShapeDtypeStruct((N, G), jnp.float32),
)


def _fc_body(w_ref, h_ref, bfc_ref, out_ref, acc_ref):
    i = pl.program_id(0)

    @pl.when(i == 0)
    def _():
        acc_ref[...] = bfc_ref[...]

    prod = w_ref[...] * h_ref[...]
    acc_ref[...] += jnp.sum(prod, axis=1, keepdims=True)

    @pl.when(i == pl.num_programs(0) - 1)
    def _():
        y = acc_ref[...]
        m = jnp.max(y)
        s = jnp.sum(jnp.exp(y - m))
        out_ref[...] = y - m - jnp.log(jnp.broadcast_to(s, (D_OUT, 1)))


_fc = pl.pallas_call(
    _fc_body,
    grid=(NG // CB,),
    in_specs=[
        pl.BlockSpec((D_OUT, CB), lambda i: (0, i)),
        pl.BlockSpec((1, CB), lambda i: (0, i)),
        pl.BlockSpec((D_OUT, 1), lambda i: (0, 0)),
    ],
    out_specs=pl.BlockSpec((D_OUT, 1), lambda i: (0, 0)),
    out_shape=jax.ShapeDtypeStruct((D_OUT, 1), jnp.float32),
    scratch_shapes=[pltpu.VMEM((D_OUT, 1), jnp.float32)],
)


def kernel(x, edge_index, W_cheb, b_cheb, W_fc, b_fc):
    x0 = x[:, 0]
    ei_flat = edge_index.reshape(2 * E)
    x_pad = jnp.concatenate([x0, jnp.zeros((N_PAD - N,), jnp.float32)])
    tx1_p, tx2_p = _get_sc_prop()(x_pad, ei_flat)
    tx1 = tx1_p[:N]
    tx2h = tx2_p[:N]
    t_mat = jnp.concatenate([x, tx1[:, None], tx2h[:, None]], axis=1)
    wc = W_cheb.reshape(3, G)
    wc_eff = jnp.stack([wc[0] - wc[2], wc[1], 2.0 * wc[2]], axis=0)
    bch = b_cheb.reshape(1, G)
    h = _hmat(t_mat, wc_eff, bch)
    h_row = h.reshape(1, NG)
    yv = _fc(W_fc, h_row, b_fc.reshape(D_OUT, 1))
    return yv[:, 0]


# speedup vs baseline: 1.0156x; 1.0016x over previous
import functools

import jax
import jax.numpy as jnp
from jax import lax
from jax.experimental import pallas as pl
from jax.experimental.pallas import tpu as pltpu
from jax.experimental.pallas import tpu_sc as plsc

N = 10000
E = 320000
G = 128
D_OUT = 10

NTILES = 16
LANES = 16
N_PAD = 10240
E_PER_TILE = E // NTILES
STRIP = N_PAD // NTILES


def _rsqrt16(d):
    i = plsc.bitcast(d, jnp.int32)
    i = jnp.int32(0x5F3759DF) - lax.shift_right_logical(i, 1)
    y = plsc.bitcast(i, jnp.float32)
    for _ in range(3):
        y = y * (1.5 - 0.5 * d * y * y)
    return jnp.where(d > 0.0, y, 0.0)


def _sc_body(x_hbm, ei_hbm, tx1_hbm, tx2_hbm,
             src_v, dst_v, dinv_v, aux_v, acc_v, rbuf_v,
             strip_v, dstrip_v, xstrip_v, sem,
             sh_all, sh_node, sh_node2):
    t = lax.axis_index("s")
    e0 = t * E_PER_TILE
    cp1 = pltpu.async_copy(ei_hbm.at[pl.ds(e0, E_PER_TILE)], src_v, sem)
    cp2 = pltpu.async_copy(ei_hbm.at[pl.ds(E + e0, E_PER_TILE)], dst_v, sem)
    cp3 = pltpu.async_copy(x_hbm.at[pl.ds(t * STRIP, STRIP)], xstrip_v, sem)

    zeros = jnp.zeros((LANES,), jnp.float32)

    def zero_acc():
        @plsc.parallel_loop(0, N_PAD, LANES, unroll=8)
        def _(i):
            acc_v[pl.ds(i, LANES)] = zeros

    def publish_and_reduce_strip():
        pltpu.sync_copy(acc_v, sh_all.at[pl.ds(t * N_PAD, N_PAD)])
        plsc.subcore_barrier()
        cps = [pltpu.async_copy(
                   sh_all.at[pl.ds(j * N_PAD + t * STRIP, STRIP)],
                   rbuf_v.at[pl.ds(j * STRIP, STRIP)], sem)
               for j in range(NTILES)]
        for cp in cps:
            cp.wait()

        @plsc.parallel_loop(0, STRIP, LANES, unroll=4)
        def _(i):
            v = rbuf_v[pl.ds(i, LANES)]
            for j in range(1, NTILES):
                v = v + rbuf_v[pl.ds(j * STRIP + i, LANES)]
            strip_v[pl.ds(i, LANES)] = v

    zero_acc()
    cp1.wait()
    cp2.wait()
    cp3.wait()
    ones = jnp.full((LANES,), 1.0, jnp.float32)

    @plsc.parallel_loop(0, E_PER_TILE, LANES, unroll=8)
    def _(i):
        s = src_v[pl.ds(i, LANES)]
        plsc.addupdate_scatter(acc_v, [s], ones)

    publish_and_reduce_strip()

    @plsc.parallel_loop(0, STRIP, LANES, unroll=4)
    def _(i):
        dv = _rsqrt16(strip_v[pl.ds(i, LANES)])
        dstrip_v[pl.ds(i, LANES)] = dv
        strip_v[pl.ds(i, LANES)] = dv * xstrip_v[pl.ds(i, LANES)]

    pltpu.sync_copy(dstrip_v, sh_node2.at[pl.ds(t * STRIP, STRIP)])
    pltpu.sync_copy(strip_v, sh_node.at[pl.ds(t * STRIP, STRIP)])
    plsc.subcore_barrier()
    cp4 = pltpu.async_copy(sh_node2, dinv_v, sem)
    cp5 = pltpu.async_copy(sh_node, aux_v, sem)

    zero_acc()
    cp4.wait()
    cp5.wait()

    @plsc.parallel_loop(0, E_PER_TILE, LANES, unroll=16)
    def _(i):
        s = src_v[pl.ds(i, LANES)]
        d = dst_v[pl.ds(i, LANES)]
        a = plsc.load_gather(aux_v, [s])
        b = plsc.load_gather(dinv_v, [d])
        plsc.addupdate_scatter(acc_v, [d], a * b)

    publish_and_reduce_strip()

    @plsc.parallel_loop(0, STRIP, LANES, unroll=4)
    def _(i):
        v = -strip_v[pl.ds(i, LANES)]
        strip_v[pl.ds(i, LANES)] = v
        dstrip_v[pl.ds(i, LANES)] = dstrip_v[pl.ds(i, LANES)] * v

    cp6 = pltpu.async_copy(strip_v, tx1_hbm.at[pl.ds(t * STRIP, STRIP)], sem)
    pltpu.sync_copy(dstrip_v, sh_node.at[pl.ds(t * STRIP, STRIP)])
    plsc.subcore_barrier()
    cp7 = pltpu.async_copy(sh_node, aux_v, sem)

    zero_acc()
    cp6.wait()
    cp7.wait()

    @plsc.parallel_loop(0, E_PER_TILE, LANES, unroll=16)
    def _(i):
        s = src_v[pl.ds(i, LANES)]
        d = dst_v[pl.ds(i, LANES)]
        a = plsc.load_gather(aux_v, [s])
        b = plsc.load_gather(dinv_v, [d])
        plsc.addupdate_scatter(acc_v, [d], a * b)

    publish_and_reduce_strip()

    @plsc.parallel_loop(0, STRIP, LANES, unroll=4)
    def _(i):
        strip_v[pl.ds(i, LANES)] = -strip_v[pl.ds(i, LANES)]

    pltpu.sync_copy(strip_v, tx2_hbm.at[pl.ds(t * STRIP, STRIP)])


@functools.cache
def _get_sc_prop():
    return pl.kernel(
        _sc_body,
        out_type=[jax.ShapeDtypeStruct((N_PAD,), jnp.float32),
                  jax.ShapeDtypeStruct((N_PAD,), jnp.float32)],
        mesh=plsc.VectorSubcoreMesh(
            core_axis_name="c", subcore_axis_name="s",
            num_cores=1, num_subcores=NTILES),
        scratch_types=[
            pltpu.VMEM((E_PER_TILE,), jnp.int32),
            pltpu.VMEM((E_PER_TILE,), jnp.int32),
            pltpu.VMEM((N_PAD,), jnp.float32),
            pltpu.VMEM((N_PAD,), jnp.float32),
            pltpu.VMEM((N_PAD,), jnp.float32),
            pltpu.VMEM((N_PAD,), jnp.float32),
            pltpu.VMEM((STRIP,), jnp.float32),
            pltpu.VMEM((STRIP,), jnp.float32),
            pltpu.VMEM((STRIP,), jnp.float32),
            pltpu.SemaphoreType.DMA,
            pltpu.VMEM_SHARED((NTILES * N_PAD,), jnp.float32),
            pltpu.VMEM_SHARED((N_PAD,), jnp.float32),
            pltpu.VMEM_SHARED((N_PAD,), jnp.float32),
        ],
        compiler_params=pltpu.CompilerParams(needs_layout_passes=False),
    )


BN = 2000
NG = N * G
BF = 1000
CB = BF * G


def _hmat_body(t_ref, wc_ref, bch_ref, h_ref):
    h = (t_ref[:, 0:1] * wc_ref[0:1, :]
         + t_ref[:, 1:2] * wc_ref[1:2, :]
         + t_ref[:, 2:3] * wc_ref[2:3, :]
         + bch_ref[0:1, :])
    h_ref[...] = jnp.maximum(h, 0.0)


_hmat = pl.pallas_call(
    _hmat_body,
    grid=(N // BN,),
    in_specs=[
        pl.BlockSpec((BN, 3), lambda i: (i, 0)),
        pl.BlockSpec((3, G), lambda i: (0, 0)),
        pl.BlockSpec((1, G), lambda i: (0, 0)),
    ],
    out_specs=pl.BlockSpec((BN, G), lambda i: (i, 0)),
    out_shape=jax.ShapeDtypeStruct((N, G), jnp.float32),
)


def _fc_body(w_ref, h_ref, bfc_ref, out_ref, acc_ref):
    i = pl.program_id(0)

    @pl.when(i == 0)
    def _():
        acc_ref[...] = bfc_ref[...]

    prod = w_ref[...] * h_ref[...]
    acc_ref[...] += jnp.sum(prod, axis=1, keepdims=True)

    @pl.when(i == pl.num_programs(0) - 1)
    def _():
        y = acc_ref[...]
        m = jnp.max(y)
        s = jnp.sum(jnp.exp(y - m))
        out_ref[...] = y - m - jnp.log(jnp.broadcast_to(s, (D_OUT, 1)))


_fc = pl.pallas_call(
    _fc_body,
    grid=(NG // CB,),
    in_specs=[
        pl.BlockSpec((D_OUT, CB), lambda i: (0, i)),
        pl.BlockSpec((1, CB), lambda i: (0, i)),
        pl.BlockSpec((D_OUT, 1), lambda i: (0, 0)),
    ],
    out_specs=pl.BlockSpec((D_OUT, 1), lambda i: (0, 0)),
    out_shape=jax.ShapeDtypeStruct((D_OUT, 1), jnp.float32),
    scratch_shapes=[pltpu.VMEM((D_OUT, 1), jnp.float32)],
)


def kernel(x, edge_index, W_cheb, b_cheb, W_fc, b_fc):
    x0 = x[:, 0]
    ei_flat = edge_index.reshape(2 * E)
    x_pad = jnp.concatenate([x0, jnp.zeros((N_PAD - N,), jnp.float32)])
    tx1_p, tx2_p = _get_sc_prop()(x_pad, ei_flat)
    tx1 = tx1_p[:N]
    tx2h = tx2_p[:N]
    t_mat = jnp.concatenate([x, tx1[:, None], tx2h[:, None]], axis=1)
    wc = W_cheb.reshape(3, G)
    wc_eff = jnp.stack([wc[0] - wc[2], wc[1], 2.0 * wc[2]], axis=0)
    bch = b_cheb.reshape(1, G)
    h = _hmat(t_mat, wc_eff, bch)
    h_row = h.reshape(1, NG)
    yv = _fc(W_fc, h_row, b_fc.reshape(D_OUT, 1))
    return yv[:, 0]
